# Initial kernel scaffold; baseline (speedup 1.0000x reference)
import jax
import jax.numpy as jnp
from jax.experimental import pallas as pl


def kernel(h, edge_index, t1_w, t1_b, gate2_w, gate2_b, gate3_w, gate3_b, tmu_w, tmu_b, tls_w, tls_b):
    raise NotImplementedError("write your pallas kernel here")


# speedup vs baseline: 2.0251x; 2.0251x over previous
import functools

import jax
import jax.numpy as jnp
from jax import lax
from jax.experimental import pallas as pl
from jax.experimental.pallas import tpu as pltpu
from jax.experimental.pallas import tpu_sc as plsc

N = 10000
E = 160000
IN_DIM = 256
HID = 512
OUT_DIM = 256
EPS = 0.3

NP_ = 10240
NC = 2
NS = 16
NW = NC * NS
EP_W = 5120
EP = NW * EP_W
NB = 40
BE = 128
DC = 32
NCH = HID // DC
RB = 256

f32 = jnp.float32
i32 = jnp.int32


def _tc_front_body(h_ref, w_ref, b_ref, g_ref, gb_ref, h0_ref, proj_ref):
    acc = jnp.dot(h_ref[...], w_ref[...], preferred_element_type=f32)
    h0 = jnp.maximum(acc + b_ref[...], 0.0)
    h0_ref[...] = h0
    proj_ref[...] = jnp.dot(h0, g_ref[...], preferred_element_type=f32) + gb_ref[...]


def _tc_front(hp, t1_w, t1_b, G, gbias):
    grid = (NP_ // RB,)
    return pl.pallas_call(
        _tc_front_body,
        grid=grid,
        in_specs=[
            pl.BlockSpec((RB, IN_DIM), lambda i: (i, 0)),
            pl.BlockSpec((IN_DIM, HID), lambda i: (0, 0)),
            pl.BlockSpec((1, HID), lambda i: (0, 0)),
            pl.BlockSpec((HID, 128), lambda i: (0, 0)),
            pl.BlockSpec((1, 128), lambda i: (0, 0)),
        ],
        out_specs=[
            pl.BlockSpec((RB, HID), lambda i: (i, 0)),
            pl.BlockSpec((RB, 128), lambda i: (i, 0)),
        ],
        out_shape=[
            jax.ShapeDtypeStruct((NP_, HID), f32),
            jax.ShapeDtypeStruct((NP_, 128), f32),
        ],
    )(hp, t1_w, t1_b, G, gbias)


def _tc_back_body(h0_ref, z2a_ref, z2b_ref, z3a_ref, z3b_ref,
                  wmu_ref, bmu_ref, wls_ref, bls_ref, mu_ref, ls_ref):
    base = EPS * h0_ref[...]
    hm = base + z2a_ref[...] + z2b_ref[...]
    mu_ref[...] = jnp.dot(hm, wmu_ref[...], preferred_element_type=f32) + bmu_ref[...]
    hl = base + z3a_ref[...] + z3b_ref[...]
    ls_ref[...] = jnp.dot(hl, wls_ref[...], preferred_element_type=f32) + bls_ref[...]


def _tc_back(h0, z2a, z2b, z3a, z3b, tmu_w, tmu_b, tls_w, tls_b):
    grid = (NP_ // RB,)
    row = lambda i: (i, 0)
    full = lambda i: (0, 0)
    return pl.pallas_call(
        _tc_back_body,
        grid=grid,
        in_specs=[
            pl.BlockSpec((RB, HID), row),
            pl.BlockSpec((RB, HID), row),
            pl.BlockSpec((RB, HID), row),
            pl.BlockSpec((RB, HID), row),
            pl.BlockSpec((RB, HID), row),
            pl.BlockSpec((HID, OUT_DIM), full),
            pl.BlockSpec((1, OUT_DIM), full),
            pl.BlockSpec((HID, OUT_DIM), full),
            pl.BlockSpec((1, OUT_DIM), full),
        ],
        out_specs=[
            pl.BlockSpec((RB, OUT_DIM), row),
            pl.BlockSpec((RB, OUT_DIM), row),
        ],
        out_shape=[
            jax.ShapeDtypeStruct((NP_, OUT_DIM), f32),
            jax.ShapeDtypeStruct((NP_, OUT_DIM), f32),
        ],
    )(h0, z2a, z2b, z3a, z3b, tmu_w, tmu_b, tls_w, tls_b)


def _sc_body(src3_h, dst3_h,
             q2d_h, q2s_h, q3d_h, q3s_h, h0r_h,
             z2_o, z3_o,
             qd_v, qs_v, dvec_v, hist2_v,
             src2_v, dst2_v, e2_v, e3_v,
             idx_v, rows_v, st2_v, st3_v, iota_v,
             a2_s, a3_s, dg_s):
    ci = lax.axis_index("c")
    si = lax.axis_index("s")
    wid = ci * NS + si

    zv16 = jnp.zeros((16,), f32)

    @pl.loop(0, 5)
    def _(k):
        for m in range(8):
            iota_v[k, pl.ds(m * 16, 16)] = (
                lax.iota(i32, 16) + (k * 128 + m * 16))

    @pl.loop(0, NP_ // 16)
    def _(r):
        dvec_v[pl.ds(r * 16, 16)] = zv16

    @pl.loop(0, NP_ // 16)
    def _(r):
        hist2_v[r] = zv16

    @pl.when(si == 0)
    def _():
        pltpu.sync_copy(hist2_v, dg_s)
    plsc.subcore_barrier()

    ones16 = jnp.ones((16,), f32)
    for half in range(2):
        pltpu.sync_copy(dst3_h.at[si * 2 + half], dst2_v)

        @pl.loop(0, NB)
        def _(b):
            for m in range(8):
                t = dst2_v[b, pl.ds(m * 16, 16)]
                plsc.addupdate_scatter(dvec_v, [t], ones16)

    @pl.loop(0, NP_ // 16)
    def _(r):
        hist2_v[r] = dvec_v[pl.ds(r * 16, 16)]

    @pl.loop(0, 5)
    def _(k):
        pltpu.sync_copy(hist2_v.at[pl.ds(k * 128, 128)],
                        dg_s.at[iota_v.at[k]], add=True)
    plsc.subcore_barrier()

    pltpu.sync_copy(dg_s, hist2_v)

    @pl.loop(0, NP_ // 16)
    def _(r):
        dg = hist2_v[r]
        x = jnp.maximum(dg, 1.0)
        bi = plsc.bitcast(x, i32)
        yi = 1597463007 - lax.shift_right_logical(bi, 1)
        y = plsc.bitcast(yi, f32)
        for _ in range(3):
            y = y * (1.5 - 0.5 * x * y * y)
        dvec_v[pl.ds(r * 16, 16)] = jnp.where(dg > 0.0, y, 0.0)

    pltpu.sync_copy(src3_h.at[wid], src2_v)
    pltpu.sync_copy(dst3_h.at[wid], dst2_v)

    def tanh16(x):
        ax = jnp.abs(x)
        u = jnp.exp(-2.0 * ax)
        th = (1.0 - u) / (1.0 + u)
        return jnp.where(x < 0.0, -th, th)

    for qd_h, qs_h, e_v in ((q2d_h, q2s_h, e2_v), (q3d_h, q3s_h, e3_v)):
        pltpu.sync_copy(qd_h, qd_v)
        pltpu.sync_copy(qs_h, qs_v)

        @pl.loop(0, NB)
        def _(b):
            for m in range(8):
                s = src2_v[b, pl.ds(m * 16, 16)]
                t = dst2_v[b, pl.ds(m * 16, 16)]
                dd = (plsc.load_gather(dvec_v, [s]) *
                      plsc.load_gather(dvec_v, [t]))
                x = (plsc.load_gather(qd_v, [t]) +
                     plsc.load_gather(qs_v, [s]))
                e_v[pl.ds(b * 128 + m * 16, 16)] = tanh16(x) * dd

    @pl.loop(0, NCH)
    def _(c):
        @pl.loop(0, BE)
        def _(j):
            st2_v[j, pl.ds(0, 16)] = zv16
            st2_v[j, pl.ds(16, 16)] = zv16
        for k in range(5):
            pltpu.sync_copy(st2_v, a2_s.at[pl.ds(si * 640 + k * 128, 128)])
            pltpu.sync_copy(st2_v, a3_s.at[pl.ds(si * 640 + k * 128, 128)])
        plsc.subcore_barrier()

        @pl.loop(0, NB)
        def _(b):
            for m in range(8):
                idx_v[pl.ds(m * 16, 16)] = (
                    src2_v[b, pl.ds(m * 16, 16)] + c * NP_)
            pltpu.sync_copy(h0r_h.at[idx_v], rows_v)

            @pl.loop(0, BE)
            def _(j):
                eidx = jnp.full((16,), b * 128 + j, i32)
                e2b = plsc.load_gather(e2_v, [eidx])
                e3b = plsc.load_gather(e3_v, [eidx])
                for k in range(2):
                    r = rows_v[j, pl.ds(k * 16, 16)]
                    st2_v[j, pl.ds(k * 16, 16)] = r * e2b
                    st3_v[j, pl.ds(k * 16, 16)] = r * e3b

            pltpu.sync_copy(st2_v, a2_s.at[dst2_v.at[b]], add=True)
            pltpu.sync_copy(st3_v, a3_s.at[dst2_v.at[b]], add=True)

        plsc.subcore_barrier()
        for k in range(5):
            r0 = si * 640 + k * 128
            pltpu.sync_copy(a2_s.at[pl.ds(r0, 128)],
                            z2_o.at[ci, c, pl.ds(r0, 128)])
            pltpu.sync_copy(a3_s.at[pl.ds(r0, 128)],
                            z3_o.at[ci, c, pl.ds(r0, 128)])
        plsc.subcore_barrier()


def _sc_edge(src3, dst3, q2d, q2s, q3d, q3s, h0r):
    mesh = plsc.VectorSubcoreMesh(core_axis_name="c", subcore_axis_name="s")
    kfn = pl.kernel(
        _sc_body,
        out_type=[
            jax.ShapeDtypeStruct((NC, NCH, NP_, DC), f32),
            jax.ShapeDtypeStruct((NC, NCH, NP_, DC), f32),
        ],
        mesh=mesh,
        compiler_params=pltpu.CompilerParams(
            needs_layout_passes=False, use_tc_tiling_on_sc=False),
        scratch_types=[
            pltpu.VMEM((NP_,), f32),
            pltpu.VMEM((NP_,), f32),
            pltpu.VMEM((NP_,), f32),
            pltpu.VMEM((NP_ // 16, 16), f32),
            pltpu.VMEM((NB, BE), i32),
            pltpu.VMEM((NB, BE), i32),
            pltpu.VMEM((EP_W,), f32),
            pltpu.VMEM((EP_W,), f32),
            pltpu.VMEM((BE,), i32),
            pltpu.VMEM((BE, DC), f32),
            pltpu.VMEM((BE, DC), f32),
            pltpu.VMEM((BE, DC), f32),
            pltpu.VMEM((5, 128), i32),
            pltpu.VMEM_SHARED((NP_, DC), f32),
            pltpu.VMEM_SHARED((NP_, DC), f32),
            pltpu.VMEM_SHARED((NP_ // 16, 16), f32),
        ],
    )
    return kfn(src3, dst3, q2d, q2s, q3d, q3s, h0r)


def kernel(h, edge_index, t1_w, t1_b, gate2_w, gate2_b, gate3_w, gate3_b,
           tmu_w, tmu_b, tls_w, tls_b):
    hp = jnp.zeros((NP_, IN_DIM), f32).at[:N].set(h)

    G = jnp.zeros((HID, 128), f32)
    G = G.at[:, 0].set(gate2_w[-1, :HID, 0])
    G = G.at[:, 1].set(gate2_w[-1, HID:, 0])
    G = G.at[:, 2].set(gate3_w[-1, :HID, 0])
    G = G.at[:, 3].set(gate3_w[-1, HID:, 0])
    gbias = jnp.zeros((1, 128), f32)
    gbias = gbias.at[0, 0].set(gate2_b[-1, 0])
    gbias = gbias.at[0, 2].set(gate3_b[-1, 0])

    h0, proj = _tc_front(hp, t1_w, t1_b.reshape(1, HID), G, gbias)

    h0r = h0.reshape(NP_, NCH, DC).transpose(1, 0, 2).reshape(NCH * NP_, DC)
    q2d = proj[:, 0]
    q2s = proj[:, 1]
    q3d = proj[:, 2]
    q3s = proj[:, 3]

    src = edge_index[0].astype(i32)
    dst = edge_index[1].astype(i32)
    srcp = jnp.full((EP,), N, i32).at[:E].set(src)
    dstp = jnp.full((EP,), N, i32).at[:E].set(dst)

    z2p, z3p = _sc_edge(
        srcp.reshape(NW, NB, BE),
        dstp.reshape(NW, NB, BE),
        q2d, q2s, q3d, q3s, h0r)

    z2a = z2p[0].transpose(1, 0, 2).reshape(NP_, HID)
    z2b = z2p[1].transpose(1, 0, 2).reshape(NP_, HID)
    z3a = z3p[0].transpose(1, 0, 2).reshape(NP_, HID)
    z3b = z3p[1].transpose(1, 0, 2).reshape(NP_, HID)

    mu, ls = _tc_back(h0, z2a, z2b, z3a, z3b,
                      tmu_w, tmu_b.reshape(1, OUT_DIM),
                      tls_w, tls_b.reshape(1, OUT_DIM))
    return (mu[:N], ls[:N])


# speedup vs baseline: 3.6528x; 1.8037x over previous
import functools

import jax
import jax.numpy as jnp
from jax import lax
from jax.experimental import pallas as pl
from jax.experimental.pallas import tpu as pltpu
from jax.experimental.pallas import tpu_sc as plsc

N = 10000
E = 160000
IN_DIM = 256
HID = 512
OUT_DIM = 256
EPS = 0.3

NP_ = 10240
NC = 2
NS = 16
NW = NC * NS
EP_W = 5120
EP = NW * EP_W
NB = 40
BE = 128
DC = 32
NCH = HID // DC
RB = 256

f32 = jnp.float32
i32 = jnp.int32


def _tc_front_body(h_ref, w_ref, b_ref, g_ref, gb_ref, h0_ref, proj_ref):
    acc = jnp.dot(h_ref[...], w_ref[...], preferred_element_type=f32)
    h0 = jnp.maximum(acc + b_ref[...], 0.0)
    h0_ref[...] = h0
    proj_ref[...] = jnp.dot(h0, g_ref[...], preferred_element_type=f32) + gb_ref[...]


def _tc_front(hp, t1_w, t1_b, G, gbias):
    grid = (NP_ // RB,)
    return pl.pallas_call(
        _tc_front_body,
        grid=grid,
        in_specs=[
            pl.BlockSpec((RB, IN_DIM), lambda i: (i, 0)),
            pl.BlockSpec((IN_DIM, HID), lambda i: (0, 0)),
            pl.BlockSpec((1, HID), lambda i: (0, 0)),
            pl.BlockSpec((HID, 128), lambda i: (0, 0)),
            pl.BlockSpec((1, 128), lambda i: (0, 0)),
        ],
        out_specs=[
            pl.BlockSpec((RB, HID), lambda i: (i, 0)),
            pl.BlockSpec((RB, 128), lambda i: (i, 0)),
        ],
        out_shape=[
            jax.ShapeDtypeStruct((NP_, HID), f32),
            jax.ShapeDtypeStruct((NP_, 128), f32),
        ],
    )(hp, t1_w, t1_b, G, gbias)


def _tc_back_body(h0_ref, z2a_ref, z2b_ref, z3a_ref, z3b_ref,
                  wmu_ref, bmu_ref, wls_ref, bls_ref, mu_ref, ls_ref):
    base = EPS * h0_ref[...]
    hm = base + z2a_ref[...] + z2b_ref[...]
    mu_ref[...] = jnp.dot(hm, wmu_ref[...], preferred_element_type=f32) + bmu_ref[...]
    hl = base + z3a_ref[...] + z3b_ref[...]
    ls_ref[...] = jnp.dot(hl, wls_ref[...], preferred_element_type=f32) + bls_ref[...]


def _tc_back(h0, z2a, z2b, z3a, z3b, tmu_w, tmu_b, tls_w, tls_b):
    grid = (NP_ // RB,)
    row = lambda i: (i, 0)
    full = lambda i: (0, 0)
    return pl.pallas_call(
        _tc_back_body,
        grid=grid,
        in_specs=[
            pl.BlockSpec((RB, HID), row),
            pl.BlockSpec((RB, HID), row),
            pl.BlockSpec((RB, HID), row),
            pl.BlockSpec((RB, HID), row),
            pl.BlockSpec((RB, HID), row),
            pl.BlockSpec((HID, OUT_DIM), full),
            pl.BlockSpec((1, OUT_DIM), full),
            pl.BlockSpec((HID, OUT_DIM), full),
            pl.BlockSpec((1, OUT_DIM), full),
        ],
        out_specs=[
            pl.BlockSpec((RB, OUT_DIM), row),
            pl.BlockSpec((RB, OUT_DIM), row),
        ],
        out_shape=[
            jax.ShapeDtypeStruct((NP_, OUT_DIM), f32),
            jax.ShapeDtypeStruct((NP_, OUT_DIM), f32),
        ],
    )(h0, z2a, z2b, z3a, z3b, tmu_w, tmu_b, tls_w, tls_b)


def _sc_body(src3_h, dst3_h,
             q2d_h, q2s_h, q3d_h, q3s_h, h0r_h,
             z2_o, z3_o,
             qd_v, qs_v, dvec_v, hist2_v,
             src2_v, dst2_v, e2_v, e3_v,
             idx0_v, idx1_v, rows0_v, rows1_v,
             st20_v, st21_v, st30_v, st31_v, iota_v,
             gsem, ssem2, ssem3,
             a2_s, a3_s, dg_s):
    ci = lax.axis_index("c")
    si = lax.axis_index("s")
    wid = ci * NS + si

    zv16 = jnp.zeros((16,), f32)

    @pl.loop(0, 5)
    def _(k):
        for m in range(8):
            iota_v[k, pl.ds(m * 16, 16)] = (
                lax.iota(i32, 16) + (k * 128 + m * 16))

    @pl.loop(0, NP_ // 16)
    def _(r):
        dvec_v[pl.ds(r * 16, 16)] = zv16

    @pl.loop(0, NP_ // 16)
    def _(r):
        hist2_v[r] = zv16

    @pl.when(si == 0)
    def _():
        pltpu.sync_copy(hist2_v, dg_s)
    plsc.subcore_barrier()

    ones16 = jnp.ones((16,), f32)
    for half in range(2):
        pltpu.sync_copy(dst3_h.at[si * 2 + half], dst2_v)

        @pl.loop(0, NB)
        def _(b):
            for m in range(8):
                t = dst2_v[b, pl.ds(m * 16, 16)]
                plsc.addupdate_scatter(dvec_v, [t], ones16)

    @pl.loop(0, NP_ // 16)
    def _(r):
        hist2_v[r] = dvec_v[pl.ds(r * 16, 16)]

    @pl.loop(0, 5)
    def _(k):
        pltpu.sync_copy(hist2_v.at[pl.ds(k * 128, 128)],
                        dg_s.at[iota_v.at[k]], add=True)
    plsc.subcore_barrier()

    pltpu.sync_copy(dg_s, hist2_v)

    @pl.loop(0, NP_ // 16)
    def _(r):
        dg = hist2_v[r]
        x = jnp.maximum(dg, 1.0)
        bi = plsc.bitcast(x, i32)
        yi = 1597463007 - lax.shift_right_logical(bi, 1)
        y = plsc.bitcast(yi, f32)
        for _ in range(3):
            y = y * (1.5 - 0.5 * x * y * y)
        dvec_v[pl.ds(r * 16, 16)] = jnp.where(dg > 0.0, y, 0.0)

    pltpu.sync_copy(src3_h.at[wid], src2_v)
    pltpu.sync_copy(dst3_h.at[wid], dst2_v)

    def tanh16(x):
        ax = jnp.abs(x)
        u = jnp.exp(-2.0 * ax)
        th = (1.0 - u) / (1.0 + u)
        return jnp.where(x < 0.0, -th, th)

    for qd_h, qs_h, e_v in ((q2d_h, q2s_h, e2_v), (q3d_h, q3s_h, e3_v)):
        pltpu.sync_copy(qd_h, qd_v)
        pltpu.sync_copy(qs_h, qs_v)

        @pl.loop(0, NB)
        def _(b):
            for m in range(8):
                s = src2_v[b, pl.ds(m * 16, 16)]
                t = dst2_v[b, pl.ds(m * 16, 16)]
                dd = (plsc.load_gather(dvec_v, [s]) *
                      plsc.load_gather(dvec_v, [t]))
                x = (plsc.load_gather(qd_v, [t]) +
                     plsc.load_gather(qs_v, [s]))
                e_v[pl.ds(b * 128 + m * 16, 16)] = tanh16(x) * dd

    rowsb = (rows0_v, rows1_v)
    st2b = (st20_v, st21_v)
    st3b = (st30_v, st31_v)
    idxb = (idx0_v, idx1_v)

    @pl.loop(0, NCH)
    def _(c):
        def gissue(par, b):
            for m in range(8):
                idxb[par][pl.ds(m * 16, 16)] = (
                    src2_v[b, pl.ds(m * 16, 16)] + c * NP_)
            pltpu.async_copy(h0r_h.at[idxb[par]], rowsb[par], gsem[par])

        @pl.loop(0, BE, unroll=4)
        def _(j):
            st20_v[j, pl.ds(0, 16)] = zv16
            st20_v[j, pl.ds(16, 16)] = zv16
        for k in range(5):
            pltpu.sync_copy(st20_v, a2_s.at[pl.ds(si * 640 + k * 128, 128)])
            pltpu.sync_copy(st20_v, a3_s.at[pl.ds(si * 640 + k * 128, 128)])
        plsc.subcore_barrier()

        gissue(0, 0)
        gissue(1, 1)

        @pl.loop(0, NB, step=2)
        def _(i):
            for par in range(2):
                b = i + par
                pltpu.make_async_copy(
                    h0r_h.at[idxb[par]], rowsb[par], gsem[par]).wait()

                @pl.when(b >= 2)
                def _():
                    pltpu.make_async_copy(
                        st2b[par], a2_s.at[dst2_v.at[b - 2]],
                        ssem2[par]).wait()
                    pltpu.make_async_copy(
                        st3b[par], a3_s.at[dst2_v.at[b - 2]],
                        ssem3[par]).wait()

                @pl.loop(0, BE, unroll=4)
                def _(j):
                    eidx = jnp.full((16,), b * BE + j, i32)
                    e2v = plsc.load_gather(e2_v, [eidx])
                    e3v = plsc.load_gather(e3_v, [eidx])
                    for k in range(2):
                        r = rowsb[par][j, pl.ds(k * 16, 16)]
                        st2b[par][j, pl.ds(k * 16, 16)] = r * e2v
                        st3b[par][j, pl.ds(k * 16, 16)] = r * e3v

                pltpu.async_copy(st2b[par], a2_s.at[dst2_v.at[b]],
                                 ssem2[par], add=True)
                pltpu.async_copy(st3b[par], a3_s.at[dst2_v.at[b]],
                                 ssem3[par], add=True)

                @pl.when(b + 2 < NB)
                def _():
                    gissue(par, b + 2)

        for par in range(2):
            b = NB - 2 + par
            pltpu.make_async_copy(
                st2b[par], a2_s.at[dst2_v.at[b]], ssem2[par]).wait()
            pltpu.make_async_copy(
                st3b[par], a3_s.at[dst2_v.at[b]], ssem3[par]).wait()

        plsc.subcore_barrier()
        for k in range(5):
            r0 = si * 640 + k * 128
            pltpu.sync_copy(a2_s.at[pl.ds(r0, 128)],
                            z2_o.at[ci, pl.ds(r0, 128), pl.ds(c * DC, DC)])
            pltpu.sync_copy(a3_s.at[pl.ds(r0, 128)],
                            z3_o.at[ci, pl.ds(r0, 128), pl.ds(c * DC, DC)])
        plsc.subcore_barrier()


def _sc_edge(src3, dst3, q2d, q2s, q3d, q3s, h0r):
    mesh = plsc.VectorSubcoreMesh(core_axis_name="c", subcore_axis_name="s")
    kfn = pl.kernel(
        _sc_body,
        out_type=[
            jax.ShapeDtypeStruct((NC, NP_, HID), f32),
            jax.ShapeDtypeStruct((NC, NP_, HID), f32),
        ],
        mesh=mesh,
        compiler_params=pltpu.CompilerParams(
            needs_layout_passes=False, use_tc_tiling_on_sc=False),
        scratch_types=[
            pltpu.VMEM((NP_,), f32),
            pltpu.VMEM((NP_,), f32),
            pltpu.VMEM((NP_,), f32),
            pltpu.VMEM((NP_ // 16, 16), f32),
            pltpu.VMEM((NB, BE), i32),
            pltpu.VMEM((NB, BE), i32),
            pltpu.VMEM((EP_W,), f32),
            pltpu.VMEM((EP_W,), f32),
            pltpu.VMEM((BE,), i32),
            pltpu.VMEM((BE,), i32),
            pltpu.VMEM((BE, DC), f32),
            pltpu.VMEM((BE, DC), f32),
            pltpu.VMEM((BE, DC), f32),
            pltpu.VMEM((BE, DC), f32),
            pltpu.VMEM((BE, DC), f32),
            pltpu.VMEM((BE, DC), f32),
            pltpu.VMEM((5, 128), i32),
            [pltpu.SemaphoreType.DMA, pltpu.SemaphoreType.DMA],
            [pltpu.SemaphoreType.DMA, pltpu.SemaphoreType.DMA],
            [pltpu.SemaphoreType.DMA, pltpu.SemaphoreType.DMA],
            pltpu.VMEM_SHARED((NP_, DC), f32),
            pltpu.VMEM_SHARED((NP_, DC), f32),
            pltpu.VMEM_SHARED((NP_ // 16, 16), f32),
        ],
    )
    return kfn(src3, dst3, q2d, q2s, q3d, q3s, h0r)


def kernel(h, edge_index, t1_w, t1_b, gate2_w, gate2_b, gate3_w, gate3_b,
           tmu_w, tmu_b, tls_w, tls_b):
    hp = jnp.zeros((NP_, IN_DIM), f32).at[:N].set(h)

    G = jnp.zeros((HID, 128), f32)
    G = G.at[:, 0].set(gate2_w[-1, :HID, 0])
    G = G.at[:, 1].set(gate2_w[-1, HID:, 0])
    G = G.at[:, 2].set(gate3_w[-1, :HID, 0])
    G = G.at[:, 3].set(gate3_w[-1, HID:, 0])
    gbias = jnp.zeros((1, 128), f32)
    gbias = gbias.at[0, 0].set(gate2_b[-1, 0])
    gbias = gbias.at[0, 2].set(gate3_b[-1, 0])

    h0, proj = _tc_front(hp, t1_w, t1_b.reshape(1, HID), G, gbias)

    h0r = h0.reshape(NP_, NCH, DC).transpose(1, 0, 2).reshape(NCH * NP_, DC)
    q2d = proj[:, 0]
    q2s = proj[:, 1]
    q3d = proj[:, 2]
    q3s = proj[:, 3]

    src = edge_index[0].astype(i32)
    dst = edge_index[1].astype(i32)
    srcp = jnp.full((EP,), N, i32).at[:E].set(src)
    dstp = jnp.full((EP,), N, i32).at[:E].set(dst)

    z2p, z3p = _sc_edge(
        srcp.reshape(NW, NB, BE),
        dstp.reshape(NW, NB, BE),
        q2d, q2s, q3d, q3s, h0r)

    z2a, z2b = z2p[0], z2p[1]
    z3a, z3b = z3p[0], z3p[1]

    mu, ls = _tc_back(h0, z2a, z2b, z3a, z3b,
                      tmu_w, tmu_b.reshape(1, OUT_DIM),
                      tls_w, tls_b.reshape(1, OUT_DIM))
    return (mu[:N], ls[:N])


# speedup vs baseline: 4.1660x; 1.1405x over previous
import functools

import jax
import jax.numpy as jnp
from jax import lax
from jax.experimental import pallas as pl
from jax.experimental.pallas import tpu as pltpu
from jax.experimental.pallas import tpu_sc as plsc

N = 10000
E = 160000
IN_DIM = 256
HID = 512
OUT_DIM = 256
EPS = 0.3

NP_ = 10240
NC = 2
NS = 16
NW = NC * NS
EP_W = 5120
EP = NW * EP_W
NB = 40
BE = 128
DC = 32
NCH = HID // DC
RB = 256

f32 = jnp.float32
i32 = jnp.int32


def _tc_front_body(h_ref, w_ref, b_ref, g_ref, gb_ref, h0_ref, proj_ref):
    acc = jnp.dot(h_ref[...], w_ref[...], preferred_element_type=f32)
    h0 = jnp.maximum(acc + b_ref[...], 0.0)
    h0_ref[...] = h0
    proj_ref[...] = jnp.dot(h0, g_ref[...], preferred_element_type=f32) + gb_ref[...]


def _tc_front(hp, t1_w, t1_b, G, gbias):
    grid = (NP_ // RB,)
    return pl.pallas_call(
        _tc_front_body,
        grid=grid,
        in_specs=[
            pl.BlockSpec((RB, IN_DIM), lambda i: (i, 0)),
            pl.BlockSpec((IN_DIM, HID), lambda i: (0, 0)),
            pl.BlockSpec((1, HID), lambda i: (0, 0)),
            pl.BlockSpec((HID, 128), lambda i: (0, 0)),
            pl.BlockSpec((1, 128), lambda i: (0, 0)),
        ],
        out_specs=[
            pl.BlockSpec((RB, HID), lambda i: (i, 0)),
            pl.BlockSpec((RB, 128), lambda i: (i, 0)),
        ],
        out_shape=[
            jax.ShapeDtypeStruct((NP_, HID), f32),
            jax.ShapeDtypeStruct((NP_, 128), f32),
        ],
    )(hp, t1_w, t1_b, G, gbias)


def _tc_back_body(h0_ref, z2a_ref, z2b_ref, z3a_ref, z3b_ref,
                  wmu_ref, bmu_ref, wls_ref, bls_ref, mu_ref, ls_ref):
    base = EPS * h0_ref[...]
    hm = base + z2a_ref[...] + z2b_ref[...]
    mu_ref[...] = jnp.dot(hm, wmu_ref[...], preferred_element_type=f32) + bmu_ref[...]
    hl = base + z3a_ref[...] + z3b_ref[...]
    ls_ref[...] = jnp.dot(hl, wls_ref[...], preferred_element_type=f32) + bls_ref[...]


def _tc_back(h0, z2a, z2b, z3a, z3b, tmu_w, tmu_b, tls_w, tls_b):
    grid = (NP_ // RB,)
    row = lambda i: (i, 0)
    full = lambda i: (0, 0)
    return pl.pallas_call(
        _tc_back_body,
        grid=grid,
        in_specs=[
            pl.BlockSpec((RB, HID), row),
            pl.BlockSpec((RB, HID), row),
            pl.BlockSpec((RB, HID), row),
            pl.BlockSpec((RB, HID), row),
            pl.BlockSpec((RB, HID), row),
            pl.BlockSpec((HID, OUT_DIM), full),
            pl.BlockSpec((1, OUT_DIM), full),
            pl.BlockSpec((HID, OUT_DIM), full),
            pl.BlockSpec((1, OUT_DIM), full),
        ],
        out_specs=[
            pl.BlockSpec((RB, OUT_DIM), row),
            pl.BlockSpec((RB, OUT_DIM), row),
        ],
        out_shape=[
            jax.ShapeDtypeStruct((NP_, OUT_DIM), f32),
            jax.ShapeDtypeStruct((NP_, OUT_DIM), f32),
        ],
    )(h0, z2a, z2b, z3a, z3b, tmu_w, tmu_b, tls_w, tls_b)


def _sc_body(src3_h, dst3_h,
             q2d_h, q2s_h, q3d_h, q3s_h, h0r_h,
             z2_o, z3_o,
             qd_v, qs_v, dvec_v, hist2_v,
             src2_v, dst2_v, e2_v, e3_v,
             idx0_v, idx1_v, rows0_v, rows1_v,
             st20_v, st21_v, st30_v, st31_v, iota_v,
             gsem, ssem2, ssem3,
             a2_s, a3_s, dg_s):
    ci = lax.axis_index("c")
    si = lax.axis_index("s")
    wid = ci * NS + si

    zv16 = jnp.zeros((16,), f32)

    @pl.loop(0, 5)
    def _(k):
        for m in range(8):
            iota_v[k, pl.ds(m * 16, 16)] = (
                lax.iota(i32, 16) + (k * 128 + m * 16))

    @pl.loop(0, NP_ // 16)
    def _(r):
        dvec_v[pl.ds(r * 16, 16)] = zv16

    @pl.loop(0, NP_ // 16)
    def _(r):
        hist2_v[r] = zv16

    @pl.when(si == 0)
    def _():
        pltpu.sync_copy(hist2_v, dg_s)
    plsc.subcore_barrier()

    ones16 = jnp.ones((16,), f32)
    for half in range(2):
        pltpu.sync_copy(dst3_h.at[si * 2 + half], dst2_v)

        @pl.loop(0, NB)
        def _(b):
            for m in range(8):
                t = dst2_v[b, pl.ds(m * 16, 16)]
                plsc.addupdate_scatter(dvec_v, [t], ones16)

    @pl.loop(0, NP_ // 16)
    def _(r):
        hist2_v[r] = dvec_v[pl.ds(r * 16, 16)]

    @pl.loop(0, 5)
    def _(k):
        pltpu.sync_copy(hist2_v.at[pl.ds(k * 128, 128)],
                        dg_s.at[iota_v.at[k]], add=True)
    plsc.subcore_barrier()

    pltpu.sync_copy(dg_s, hist2_v)

    @pl.loop(0, NP_ // 16)
    def _(r):
        dg = hist2_v[r]
        x = jnp.maximum(dg, 1.0)
        bi = plsc.bitcast(x, i32)
        yi = 1597463007 - lax.shift_right_logical(bi, 1)
        y = plsc.bitcast(yi, f32)
        for _ in range(3):
            y = y * (1.5 - 0.5 * x * y * y)
        dvec_v[pl.ds(r * 16, 16)] = jnp.where(dg > 0.0, y, 0.0)

    pltpu.sync_copy(src3_h.at[wid], src2_v)
    pltpu.sync_copy(dst3_h.at[wid], dst2_v)

    def tanh16(x):
        ax = jnp.abs(x)
        u = jnp.exp(-2.0 * ax)
        th = (1.0 - u) / (1.0 + u)
        return jnp.where(x < 0.0, -th, th)

    for qd_h, qs_h, e_v in ((q2d_h, q2s_h, e2_v), (q3d_h, q3s_h, e3_v)):
        pltpu.sync_copy(qd_h, qd_v)
        pltpu.sync_copy(qs_h, qs_v)

        @pl.loop(0, NB)
        def _(b):
            for m in range(8):
                s = src2_v[b, pl.ds(m * 16, 16)]
                t = dst2_v[b, pl.ds(m * 16, 16)]
                dd = (plsc.load_gather(dvec_v, [s]) *
                      plsc.load_gather(dvec_v, [t]))
                x = (plsc.load_gather(qd_v, [t]) +
                     plsc.load_gather(qs_v, [s]))
                e_v[pl.ds(b * 128 + m * 16, 16)] = tanh16(x) * dd

    rowsb = (rows0_v, rows1_v)
    st2b = (st20_v, st21_v)
    st3b = (st30_v, st31_v)
    idxb = (idx0_v, idx1_v)

    @pl.loop(0, NCH)
    def _(c):
        def gissue(par, b):
            for m in range(8):
                idxb[par][pl.ds(m * 16, 16)] = (
                    src2_v[b, pl.ds(m * 16, 16)] + c * NP_)
            pltpu.async_copy(h0r_h.at[idxb[par]], rowsb[par], gsem[par])

        @pl.loop(0, BE, unroll=4)
        def _(j):
            st20_v[j, pl.ds(0, 16)] = zv16
            st20_v[j, pl.ds(16, 16)] = zv16
        for k in range(5):
            pltpu.sync_copy(st20_v, a2_s.at[pl.ds(si * 640 + k * 128, 128)])
            pltpu.sync_copy(st20_v, a3_s.at[pl.ds(si * 640 + k * 128, 128)])
        plsc.subcore_barrier()

        gissue(0, 0)
        gissue(1, 1)

        @pl.loop(0, NB, step=2)
        def _(i):
            for par in range(2):
                b = i + par
                pltpu.make_async_copy(
                    h0r_h.at[idxb[par]], rowsb[par], gsem[par]).wait()

                @pl.when(b >= 2)
                def _():
                    pltpu.make_async_copy(
                        st2b[par], a2_s.at[dst2_v.at[b - 2]],
                        ssem2[par]).wait()
                    pltpu.make_async_copy(
                        st3b[par], a3_s.at[dst2_v.at[b - 2]],
                        ssem3[par]).wait()

                @pl.loop(0, BE // 16)
                def _(jj):
                    e2g = e2_v[pl.ds(b * BE + jj * 16, 16)]
                    e3g = e3_v[pl.ds(b * BE + jj * 16, 16)]
                    for j16 in range(16):
                        j = jj * 16 + j16
                        e2s = e2g[j16]
                        e3s = e3g[j16]
                        for k in range(2):
                            r = rowsb[par][j, pl.ds(k * 16, 16)]
                            st2b[par][j, pl.ds(k * 16, 16)] = r * e2s
                            st3b[par][j, pl.ds(k * 16, 16)] = r * e3s

                pltpu.async_copy(st2b[par], a2_s.at[dst2_v.at[b]],
                                 ssem2[par], add=True)
                pltpu.async_copy(st3b[par], a3_s.at[dst2_v.at[b]],
                                 ssem3[par], add=True)

                @pl.when(b + 2 < NB)
                def _():
                    gissue(par, b + 2)

        for par in range(2):
            b = NB - 2 + par
            pltpu.make_async_copy(
                st2b[par], a2_s.at[dst2_v.at[b]], ssem2[par]).wait()
            pltpu.make_async_copy(
                st3b[par], a3_s.at[dst2_v.at[b]], ssem3[par]).wait()

        plsc.subcore_barrier()
        for k in range(5):
            r0 = si * 640 + k * 128
            pltpu.sync_copy(a2_s.at[pl.ds(r0, 128)],
                            z2_o.at[ci, pl.ds(r0, 128), pl.ds(c * DC, DC)])
            pltpu.sync_copy(a3_s.at[pl.ds(r0, 128)],
                            z3_o.at[ci, pl.ds(r0, 128), pl.ds(c * DC, DC)])
        plsc.subcore_barrier()


def _sc_edge(src3, dst3, q2d, q2s, q3d, q3s, h0r):
    mesh = plsc.VectorSubcoreMesh(core_axis_name="c", subcore_axis_name="s")
    kfn = pl.kernel(
        _sc_body,
        out_type=[
            jax.ShapeDtypeStruct((NC, NP_, HID), f32),
            jax.ShapeDtypeStruct((NC, NP_, HID), f32),
        ],
        mesh=mesh,
        compiler_params=pltpu.CompilerParams(
            needs_layout_passes=False, use_tc_tiling_on_sc=False),
        scratch_types=[
            pltpu.VMEM((NP_,), f32),
            pltpu.VMEM((NP_,), f32),
            pltpu.VMEM((NP_,), f32),
            pltpu.VMEM((NP_ // 16, 16), f32),
            pltpu.VMEM((NB, BE), i32),
            pltpu.VMEM((NB, BE), i32),
            pltpu.VMEM((EP_W,), f32),
            pltpu.VMEM((EP_W,), f32),
            pltpu.VMEM((BE,), i32),
            pltpu.VMEM((BE,), i32),
            pltpu.VMEM((BE, DC), f32),
            pltpu.VMEM((BE, DC), f32),
            pltpu.VMEM((BE, DC), f32),
            pltpu.VMEM((BE, DC), f32),
            pltpu.VMEM((BE, DC), f32),
            pltpu.VMEM((BE, DC), f32),
            pltpu.VMEM((5, 128), i32),
            [pltpu.SemaphoreType.DMA, pltpu.SemaphoreType.DMA],
            [pltpu.SemaphoreType.DMA, pltpu.SemaphoreType.DMA],
            [pltpu.SemaphoreType.DMA, pltpu.SemaphoreType.DMA],
            pltpu.VMEM_SHARED((NP_, DC), f32),
            pltpu.VMEM_SHARED((NP_, DC), f32),
            pltpu.VMEM_SHARED((NP_ // 16, 16), f32),
        ],
    )
    return kfn(src3, dst3, q2d, q2s, q3d, q3s, h0r)


def kernel(h, edge_index, t1_w, t1_b, gate2_w, gate2_b, gate3_w, gate3_b,
           tmu_w, tmu_b, tls_w, tls_b):
    hp = jnp.zeros((NP_, IN_DIM), f32).at[:N].set(h)

    G = jnp.zeros((HID, 128), f32)
    G = G.at[:, 0].set(gate2_w[-1, :HID, 0])
    G = G.at[:, 1].set(gate2_w[-1, HID:, 0])
    G = G.at[:, 2].set(gate3_w[-1, :HID, 0])
    G = G.at[:, 3].set(gate3_w[-1, HID:, 0])
    gbias = jnp.zeros((1, 128), f32)
    gbias = gbias.at[0, 0].set(gate2_b[-1, 0])
    gbias = gbias.at[0, 2].set(gate3_b[-1, 0])

    h0, proj = _tc_front(hp, t1_w, t1_b.reshape(1, HID), G, gbias)

    h0r = h0.reshape(NP_, NCH, DC).transpose(1, 0, 2).reshape(NCH * NP_, DC)
    q2d = proj[:, 0]
    q2s = proj[:, 1]
    q3d = proj[:, 2]
    q3s = proj[:, 3]

    src = edge_index[0].astype(i32)
    dst = edge_index[1].astype(i32)
    srcp = jnp.full((EP,), N, i32).at[:E].set(src)
    dstp = jnp.full((EP,), N, i32).at[:E].set(dst)

    z2p, z3p = _sc_edge(
        srcp.reshape(NW, NB, BE),
        dstp.reshape(NW, NB, BE),
        q2d, q2s, q3d, q3s, h0r)

    z2a, z2b = z2p[0], z2p[1]
    z3a, z3b = z3p[0], z3p[1]

    mu, ls = _tc_back(h0, z2a, z2b, z3a, z3b,
                      tmu_w, tmu_b.reshape(1, OUT_DIM),
                      tls_w, tls_b.reshape(1, OUT_DIM))
    return (mu[:N], ls[:N])
